# baseline (speedup 1.0000x reference)
import functools

import jax
import jax.numpy as jnp
from jax.experimental import pallas as pl
from jax.experimental.pallas import tpu as pltpu

NUM_GRAPHS = 512
_CH = 256
_TILE = 2000
_STEPS = (25, 5, 5)


def _accumulate(x_ref, b_ref, tok_ref, Wl_ref, bl_ref, Wr_ref, br_ref,
                att_ref, acc_ref, first_step):
    i = pl.program_id(0)

    @pl.when(i == first_step)
    def _init():
        acc_ref[...] = jnp.zeros_like(acc_ref)

    x = x_ref[...]
    xl = jnp.dot(x.astype(jnp.bfloat16), Wl_ref[0].astype(jnp.bfloat16),
                 preferred_element_type=jnp.float32) + bl_ref[0]
    r = jnp.dot(tok_ref[0], Wr_ref[0],
                preferred_element_type=jnp.float32) + br_ref[0]
    z = xl + r
    att = att_ref[0]
    att_sel = jnp.where(z >= 0.0, att, 0.2 * att)
    logit = jnp.sum(z * att_sel, axis=1, keepdims=True)
    ex = jnp.exp(logit)

    b16 = b_ref[...].astype(jnp.int16)
    onehot = (b16 == jax.lax.broadcasted_iota(jnp.int16, (_TILE, NUM_GRAPHS), 1))
    ow = jnp.where(onehot, ex.astype(jnp.bfloat16), jnp.bfloat16(0.0))

    xlp = jnp.concatenate(
        [xl.astype(jnp.bfloat16), jnp.ones((_TILE, 1), jnp.bfloat16)], axis=1)
    acc_ref[...] += jax.lax.dot_general(
        ow, xlp, (((0,), (0,)), ((), ())),
        preferred_element_type=jnp.float32)


def _pool(acc_ref, bias_ref):
    den = acc_ref[:, _CH:_CH + 1]
    return acc_ref[:, :_CH] / (den + 1e-16) + bias_ref[0]


def _fused_step(x_op_ref, x_ma_ref, x_ag_ref, b_op_ref, b_ma_ref, b_ag_ref,
                tok_ref, Wl_ref, bl_ref, Wr_ref, br_ref, att_ref, bias_ref,
                W1a_ref, W1b_ref, W1c_ref, b1_ref, W2_ref, b2_ref,
                g_op_ref, g_ma_ref, g_ag_ref, graph_ref, acc_ref):
    i = pl.program_id(0)
    s0, s1, s2 = _STEPS
    e0, e1, e2 = s0, s0 + s1, s0 + s1 + s2

    @pl.when(i < e0)
    def _type0():
        _accumulate(x_op_ref, b_op_ref, tok_ref, Wl_ref, bl_ref, Wr_ref,
                    br_ref, att_ref, acc_ref, 0)

    @pl.when((i >= e0) & (i < e1))
    def _type1():
        _accumulate(x_ma_ref, b_ma_ref, tok_ref, Wl_ref, bl_ref, Wr_ref,
                    br_ref, att_ref, acc_ref, e0)

    @pl.when(i >= e1)
    def _type2():
        _accumulate(x_ag_ref, b_ag_ref, tok_ref, Wl_ref, bl_ref, Wr_ref,
                    br_ref, att_ref, acc_ref, e1)

    @pl.when(i == e0 - 1)
    def _fin0():
        g_op_ref[...] = _pool(acc_ref, bias_ref)

    @pl.when(i == e1 - 1)
    def _fin1():
        g_ma_ref[...] = _pool(acc_ref, bias_ref)

    @pl.when(i == e2 - 1)
    def _fin2():
        g_ag = _pool(acc_ref, bias_ref)
        g_ag_ref[...] = g_ag
        h = (jnp.dot(g_op_ref[...], W1a_ref[...],
                     preferred_element_type=jnp.float32)
             + jnp.dot(g_ma_ref[...], W1b_ref[...],
                       preferred_element_type=jnp.float32)
             + jnp.dot(g_ag, W1c_ref[...],
                       preferred_element_type=jnp.float32)
             + b1_ref[...])
        h = jnp.where(h >= 0.0, h, 0.01 * h)
        graph_ref[...] = jnp.dot(h, W2_ref[...],
                                 preferred_element_type=jnp.float32) + b2_ref[...]


def _type_index(i):
    s0, s1, _ = _STEPS
    return (i >= s0).astype(jnp.int32) + (i >= s0 + s1).astype(jnp.int32)


@jax.jit
def kernel(x_operation, batch_operation, tok_operation, Wl_operation,
           bl_operation, Wr_operation, br_operation, att_operation,
           bias_operation, x_machine, batch_machine, tok_machine, Wl_machine,
           bl_machine, Wr_machine, br_machine, att_machine, bias_machine,
           x_AGV, batch_AGV, tok_AGV, Wl_AGV, bl_AGV, Wr_AGV, br_AGV, att_AGV,
           bias_AGV, W1, b1, W2, b2):
    s0, s1, s2 = _STEPS
    num_steps = s0 + s1 + s2
    gc = W2.shape[0]

    def stack(*xs):
        return jnp.stack([x.reshape(1, _CH) for x in xs])

    tok_all = stack(tok_operation, tok_machine, tok_AGV)
    bl_all = stack(bl_operation, bl_machine, bl_AGV)
    br_all = stack(br_operation, br_machine, br_AGV)
    att_all = stack(att_operation, att_machine, att_AGV)
    bias_all = stack(bias_operation, bias_machine, bias_AGV)
    Wl_all = jnp.stack([Wl_operation, Wl_machine, Wl_AGV])
    Wr_all = jnp.stack([Wr_operation, Wr_machine, Wr_AGV])

    x_specs = [
        pl.BlockSpec((_TILE, _CH), lambda i: (jnp.minimum(i, s0 - 1), 0)),
        pl.BlockSpec((_TILE, _CH),
                     lambda i: (jnp.clip(i - s0, 0, s1 - 1), 0)),
        pl.BlockSpec((_TILE, _CH),
                     lambda i: (jnp.clip(i - s0 - s1, 0, s2 - 1), 0)),
    ]
    b_specs = [
        pl.BlockSpec((_TILE, 1), lambda i: (jnp.minimum(i, s0 - 1), 0)),
        pl.BlockSpec((_TILE, 1), lambda i: (jnp.clip(i - s0, 0, s1 - 1), 0)),
        pl.BlockSpec((_TILE, 1), lambda i: (jnp.clip(i - s0 - s1, 0, s2 - 1), 0)),
    ]
    row_spec = pl.BlockSpec((1, 1, _CH), lambda i: (_type_index(i), 0, 0))
    mat_spec = pl.BlockSpec((1, _CH, _CH), lambda i: (_type_index(i), 0, 0))
    const = lambda shape: pl.BlockSpec(shape, lambda i: tuple(0 for _ in shape))

    out = pl.pallas_call(
        _fused_step,
        grid=(num_steps,),
        in_specs=(x_specs + b_specs +
                  [row_spec, mat_spec, row_spec, mat_spec, row_spec, row_spec,
                   row_spec,
                   const((_CH, gc)), const((_CH, gc)), const((_CH, gc)),
                   const((1, gc)), const((gc, gc)), const((1, gc))]),
        out_specs=[
            pl.BlockSpec((NUM_GRAPHS, _CH), lambda i: (0, 0)),
            pl.BlockSpec((NUM_GRAPHS, _CH), lambda i: (0, 0)),
            pl.BlockSpec((NUM_GRAPHS, _CH), lambda i: (0, 0)),
            pl.BlockSpec((NUM_GRAPHS, gc), lambda i: (0, 0)),
        ],
        out_shape=[
            jax.ShapeDtypeStruct((NUM_GRAPHS, _CH), jnp.float32),
            jax.ShapeDtypeStruct((NUM_GRAPHS, _CH), jnp.float32),
            jax.ShapeDtypeStruct((NUM_GRAPHS, _CH), jnp.float32),
            jax.ShapeDtypeStruct((NUM_GRAPHS, gc), jnp.float32),
        ],
        scratch_shapes=[
            pltpu.VMEM((NUM_GRAPHS, _CH + 1), jnp.float32),
        ],
        compiler_params=pltpu.CompilerParams(
            dimension_semantics=("arbitrary",)),
    )(x_operation, x_machine, x_AGV,
      batch_operation.reshape(-1, 1), batch_machine.reshape(-1, 1),
      batch_AGV.reshape(-1, 1),
      tok_all, Wl_all, bl_all, Wr_all, br_all, att_all, bias_all,
      W1[:_CH], W1[_CH:2 * _CH], W1[2 * _CH:], b1.reshape(1, gc),
      W2, b2.reshape(1, gc))
    return tuple(out)


# speedup vs baseline: 1.6206x; 1.6206x over previous
import jax
import jax.numpy as jnp
from jax.experimental import pallas as pl
from jax.experimental.pallas import tpu as pltpu

NUM_GRAPHS = 512
_CH = 256
_TILE = 2000
_STEPS = (25, 5, 5)


def _accumulate(x_ref, b_ref, tok_ref, Wl_ref, bl_ref, Wr_ref, br_ref,
                att_ref, acc_ref, first_step):
    i = pl.program_id(0)

    @pl.when(i == first_step)
    def _init():
        acc_ref[...] = jnp.zeros_like(acc_ref)

    x = x_ref[...]
    xl = jnp.dot(x.astype(jnp.bfloat16), Wl_ref[...].astype(jnp.bfloat16),
                 preferred_element_type=jnp.float32) + bl_ref[...]
    r = jnp.dot(tok_ref[...], Wr_ref[...],
                preferred_element_type=jnp.float32) + br_ref[...]
    z = xl + r
    att = att_ref[...]
    att_sel = jnp.where(z >= 0.0, att, 0.2 * att)
    logit = jnp.sum(z * att_sel, axis=1, keepdims=True)
    ex_row = jax.lax.transpose(jnp.exp(logit).astype(jnp.bfloat16),
                               (1, 0))

    b16 = b_ref[0].astype(jnp.int16)
    onehot_t = (jax.lax.broadcasted_iota(jnp.int16, (NUM_GRAPHS, _TILE), 0)
                == b16)
    ow_t = jnp.where(onehot_t, ex_row, jnp.bfloat16(0.0))

    xlp = jnp.concatenate(
        [xl.astype(jnp.bfloat16), jnp.ones((_TILE, 1), jnp.bfloat16)], axis=1)
    acc_ref[...] += jnp.dot(ow_t, xlp,
                            preferred_element_type=jnp.float32)


def _pool(acc_ref, bias_ref):
    den = acc_ref[:, _CH:_CH + 1]
    return acc_ref[:, :_CH] / (den + 1e-16) + bias_ref[...]


def _fused_step(x_op_ref, x_ma_ref, x_ag_ref, b_op_ref, b_ma_ref, b_ag_ref,
                tok0_ref, Wl0_ref, bl0_ref, Wr0_ref, br0_ref, att0_ref,
                bias0_ref,
                tok1_ref, Wl1_ref, bl1_ref, Wr1_ref, br1_ref, att1_ref,
                bias1_ref,
                tok2_ref, Wl2_ref, bl2_ref, Wr2_ref, br2_ref, att2_ref,
                bias2_ref,
                W1_ref, b1_ref, W2_ref, b2_ref,
                g_op_ref, g_ma_ref, g_ag_ref, graph_ref, acc_ref):
    i = pl.program_id(0)
    s0, s1, s2 = _STEPS
    e0, e1, e2 = s0, s0 + s1, s0 + s1 + s2

    @pl.when(i < e0)
    def _type0():
        _accumulate(x_op_ref, b_op_ref, tok0_ref, Wl0_ref, bl0_ref, Wr0_ref,
                    br0_ref, att0_ref, acc_ref, 0)

    @pl.when((i >= e0) & (i < e1))
    def _type1():
        _accumulate(x_ma_ref, b_ma_ref, tok1_ref, Wl1_ref, bl1_ref, Wr1_ref,
                    br1_ref, att1_ref, acc_ref, e0)

    @pl.when(i >= e1)
    def _type2():
        _accumulate(x_ag_ref, b_ag_ref, tok2_ref, Wl2_ref, bl2_ref, Wr2_ref,
                    br2_ref, att2_ref, acc_ref, e1)

    @pl.when(i == e0 - 1)
    def _fin0():
        g_op_ref[...] = _pool(acc_ref, bias0_ref)

    @pl.when(i == e1 - 1)
    def _fin1():
        g_ma_ref[...] = _pool(acc_ref, bias1_ref)

    @pl.when(i == e2 - 1)
    def _fin2():
        g_ag = _pool(acc_ref, bias2_ref)
        g_ag_ref[...] = g_ag
        h = (jnp.dot(g_op_ref[...], W1_ref[:_CH],
                     preferred_element_type=jnp.float32)
             + jnp.dot(g_ma_ref[...], W1_ref[_CH:2 * _CH],
                       preferred_element_type=jnp.float32)
             + jnp.dot(g_ag, W1_ref[2 * _CH:],
                       preferred_element_type=jnp.float32)
             + b1_ref[...])
        h = jnp.where(h >= 0.0, h, 0.01 * h)
        graph_ref[...] = jnp.dot(h, W2_ref[...],
                                 preferred_element_type=jnp.float32) + b2_ref[...]


@jax.jit
def kernel(x_operation, batch_operation, tok_operation, Wl_operation,
           bl_operation, Wr_operation, br_operation, att_operation,
           bias_operation, x_machine, batch_machine, tok_machine, Wl_machine,
           bl_machine, Wr_machine, br_machine, att_machine, bias_machine,
           x_AGV, batch_AGV, tok_AGV, Wl_AGV, bl_AGV, Wr_AGV, br_AGV, att_AGV,
           bias_AGV, W1, b1, W2, b2):
    s0, s1, s2 = _STEPS
    num_steps = s0 + s1 + s2
    gc = W2.shape[0]

    x_specs = [
        pl.BlockSpec((_TILE, _CH), lambda i: (jnp.minimum(i, s0 - 1), 0)),
        pl.BlockSpec((_TILE, _CH),
                     lambda i: (jnp.clip(i - s0, 0, s1 - 1), 0)),
        pl.BlockSpec((_TILE, _CH),
                     lambda i: (jnp.clip(i - s0 - s1, 0, s2 - 1), 0)),
    ]
    b_specs = [
        pl.BlockSpec((1, 1, _TILE), lambda i: (jnp.minimum(i, s0 - 1), 0, 0)),
        pl.BlockSpec((1, 1, _TILE),
                     lambda i: (jnp.clip(i - s0, 0, s1 - 1), 0, 0)),
        pl.BlockSpec((1, 1, _TILE),
                     lambda i: (jnp.clip(i - s0 - s1, 0, s2 - 1), 0, 0)),
    ]
    const = lambda shape: pl.BlockSpec(shape, lambda i: tuple(0 for _ in shape))
    w_specs = [const((1, _CH)), const((_CH, _CH)), const((1, _CH)),
               const((_CH, _CH)), const((1, _CH)), const((1, _CH)),
               const((1, _CH))] * 3
    mlp_specs = [const((3 * _CH, gc)), const((1, gc)), const((gc, gc)),
                 const((1, gc))]

    out = pl.pallas_call(
        _fused_step,
        grid=(num_steps,),
        in_specs=x_specs + b_specs + w_specs + mlp_specs,
        out_specs=[
            pl.BlockSpec((NUM_GRAPHS, _CH), lambda i: (0, 0)),
            pl.BlockSpec((NUM_GRAPHS, _CH), lambda i: (0, 0)),
            pl.BlockSpec((NUM_GRAPHS, _CH), lambda i: (0, 0)),
            pl.BlockSpec((NUM_GRAPHS, gc), lambda i: (0, 0)),
        ],
        out_shape=[
            jax.ShapeDtypeStruct((NUM_GRAPHS, _CH), jnp.float32),
            jax.ShapeDtypeStruct((NUM_GRAPHS, _CH), jnp.float32),
            jax.ShapeDtypeStruct((NUM_GRAPHS, _CH), jnp.float32),
            jax.ShapeDtypeStruct((NUM_GRAPHS, gc), jnp.float32),
        ],
        scratch_shapes=[
            pltpu.VMEM((NUM_GRAPHS, _CH + 1), jnp.float32),
        ],
        compiler_params=pltpu.CompilerParams(
            dimension_semantics=("arbitrary",)),
    )(x_operation, x_machine, x_AGV,
      batch_operation.reshape(s0, 1, _TILE),
      batch_machine.reshape(s1, 1, _TILE),
      batch_AGV.reshape(s2, 1, _TILE),
      tok_operation.reshape(1, _CH), Wl_operation,
      bl_operation.reshape(1, _CH), Wr_operation,
      br_operation.reshape(1, _CH), att_operation.reshape(1, _CH),
      bias_operation.reshape(1, _CH),
      tok_machine.reshape(1, _CH), Wl_machine,
      bl_machine.reshape(1, _CH), Wr_machine,
      br_machine.reshape(1, _CH), att_machine.reshape(1, _CH),
      bias_machine.reshape(1, _CH),
      tok_AGV.reshape(1, _CH), Wl_AGV,
      bl_AGV.reshape(1, _CH), Wr_AGV,
      br_AGV.reshape(1, _CH), att_AGV.reshape(1, _CH),
      bias_AGV.reshape(1, _CH),
      W1, b1.reshape(1, gc), W2, b2.reshape(1, gc))
    return tuple(out)
